# baseline (speedup 1.0000x reference)
import functools

import jax
import jax.numpy as jnp
from jax import lax
from jax.experimental import pallas as pl
from jax.experimental.pallas import tpu as pltpu
from jax.experimental.pallas import tpu_sc as plsc

_C = 256
_HW = 4096
_N = 32768
_NC = 2
_NS = 16
_NW = _NC * _NS
_L = 16
_PER_W = _N // _NW
_NB = 64
_NBLK = _PER_W // _NB


def _sc_body(fm_hbm, s_hbm, ii_hbm, ij_hbm, out_hbm,
             i_v, ip_v, j_v, jp_v, fi_v, fip_v, sj_v, sjp_v, acc_v, sem):
    cid = lax.axis_index("c")
    sid = lax.axis_index("s")
    wid = sid * _NC + cid
    base = wid * _PER_W

    pltpu.sync_copy(ii_hbm.at[pl.ds(base, _PER_W)], i_v)
    pltpu.sync_copy(ii_hbm.at[pl.ds(_N + base, _PER_W)], ip_v)
    pltpu.sync_copy(ij_hbm.at[pl.ds(base, _PER_W)], j_v)
    pltpu.sync_copy(ij_hbm.at[pl.ds(_N + base, _PER_W)], jp_v)

    lanes = lax.iota(jnp.int32, _L)
    zero = jnp.zeros((_L,), jnp.float32)

    def block(b, acc):
        off = b * _NB
        c1 = pltpu.async_copy(fm_hbm.at[i_v.at[pl.ds(off, _NB)]], fi_v, sem)
        c2 = pltpu.async_copy(fm_hbm.at[ip_v.at[pl.ds(off, _NB)]], fip_v, sem)
        c3 = pltpu.async_copy(s_hbm.at[j_v.at[pl.ds(off, _NB)]], sj_v, sem)
        c4 = pltpu.async_copy(s_hbm.at[jp_v.at[pl.ds(off, _NB)]], sjp_v, sem)
        c1.wait()
        c2.wait()
        c3.wait()
        c4.wait()

        for g in range(_NB // _L):
            rows = g * _L + lanes

            def chan(cc, ps):
                p1, p2, p3, p4 = ps
                cols = jnp.full((_L,), cc, jnp.int32)
                a = plsc.load_gather(fi_v, [rows, cols])
                bb = plsc.load_gather(fip_v, [rows, cols])
                c_ = plsc.load_gather(sj_v, [rows, cols])
                d_ = plsc.load_gather(sjp_v, [rows, cols])
                return (p1 + a * bb, p2 + c_ * d_, p3 + a * d_, p4 + c_ * bb)

            p1, p2, p3, p4 = lax.fori_loop(0, _C, chan, (zero, zero, zero, zero))
            acc = acc + (p1 * p1 + p2 * p2 - p3 * p3 - p4 * p4)
        return acc

    acc = lax.fori_loop(0, _NBLK, block, zero)
    acc_v[...] = acc
    pltpu.sync_copy(acc_v, out_hbm.at[wid])


@functools.partial(jax.jit, static_argnums=())
def _poly_loss_sc(fm, s, idx_i, idx_j):
    mesh = plsc.VectorSubcoreMesh(core_axis_name="c", subcore_axis_name="s")
    call = pl.kernel(
        _sc_body,
        out_type=jax.ShapeDtypeStruct((_NW, _L), jnp.float32),
        mesh=mesh,
        scratch_types=[
            pltpu.VMEM((_PER_W,), jnp.int32),
            pltpu.VMEM((_PER_W,), jnp.int32),
            pltpu.VMEM((_PER_W,), jnp.int32),
            pltpu.VMEM((_PER_W,), jnp.int32),
            pltpu.VMEM((_NB, _C), jnp.float32),
            pltpu.VMEM((_NB, _C), jnp.float32),
            pltpu.VMEM((_NB, _C), jnp.float32),
            pltpu.VMEM((_NB, _C), jnp.float32),
            pltpu.VMEM((_L,), jnp.float32),
            pltpu.SemaphoreType.DMA,
        ],
        compiler_params=pltpu.CompilerParams(needs_layout_passes=False),
    )
    return call(fm, s, idx_i, idx_j)


def kernel(input, target, idx_i, idx_j):
    c = input.shape[1]
    fm = input.reshape(c, -1).T
    s = target.reshape(c, -1).T
    partials = _poly_loss_sc(fm, s, idx_i, idx_j)
    n = idx_i.shape[0] // 2
    return partials.sum() / jnp.float32(n) / jnp.float32(c * c)


# speedup vs baseline: 1.0400x; 1.0400x over previous
import functools

import jax
import jax.numpy as jnp
from jax import lax
from jax.experimental import pallas as pl
from jax.experimental.pallas import tpu as pltpu
from jax.experimental.pallas import tpu_sc as plsc

_C = 256
_HW = 4096
_N = 32768
_NC = 2
_NS = 16
_NW = _NC * _NS
_L = 16
_PER_W = _N // _NW
_NB = 32
_NBLK = _PER_W // _NB
_UNROLL = 8


def _sc_body(fm_hbm, s_hbm, ii_hbm, ij_hbm, out_hbm,
             i_v, ip_v, j_v, jp_v,
             fi0, fip0, sj0, sjp0, fi1, fip1, sj1, sjp1,
             acc_v, sem0, sem1):
    cid = lax.axis_index("c")
    sid = lax.axis_index("s")
    wid = sid * _NC + cid
    base = wid * _PER_W

    pltpu.sync_copy(ii_hbm.at[pl.ds(base, _PER_W)], i_v)
    pltpu.sync_copy(ii_hbm.at[pl.ds(_N + base, _PER_W)], ip_v)
    pltpu.sync_copy(ij_hbm.at[pl.ds(base, _PER_W)], j_v)
    pltpu.sync_copy(ij_hbm.at[pl.ds(_N + base, _PER_W)], jp_v)

    lanes = lax.iota(jnp.int32, _L)
    zero = jnp.zeros((_L,), jnp.float32)

    def fetch(blk, bufs, sem):
        off = blk * _NB
        fi, fip, sj, sjp = bufs
        h1 = pltpu.async_copy(fm_hbm.at[i_v.at[pl.ds(off, _NB)]], fi, sem)
        h2 = pltpu.async_copy(fm_hbm.at[ip_v.at[pl.ds(off, _NB)]], fip, sem)
        h3 = pltpu.async_copy(s_hbm.at[j_v.at[pl.ds(off, _NB)]], sj, sem)
        h4 = pltpu.async_copy(s_hbm.at[jp_v.at[pl.ds(off, _NB)]], sjp, sem)
        return h1, h2, h3, h4

    def drain(handles):
        for h in handles:
            h.wait()

    def compute(bufs, acc):
        fi, fip, sj, sjp = bufs
        for g in range(_NB // _L):
            rows = g * _L + lanes

            def chan(k, carry):
                p1, p2, p3, p4 = carry
                cbase = k * _UNROLL
                for _u in range(_UNROLL):
                    cols = jnp.full((_L,), cbase + _u, jnp.int32)
                    a = plsc.load_gather(fi, [rows, cols])
                    bb = plsc.load_gather(fip, [rows, cols])
                    c_ = plsc.load_gather(sj, [rows, cols])
                    d_ = plsc.load_gather(sjp, [rows, cols])
                    p1 = p1 + a * bb
                    p2 = p2 + c_ * d_
                    p3 = p3 + a * d_
                    p4 = p4 + c_ * bb
                return p1, p2, p3, p4

            init = (zero, zero, zero, zero)
            p1, p2, p3, p4 = lax.fori_loop(0, _C // _UNROLL, chan, init)
            acc = acc + (p1 * p1 + p2 * p2 - p3 * p3 - p4 * p4)
        return acc

    bufs0 = (fi0, fip0, sj0, sjp0)
    bufs1 = (fi1, fip1, sj1, sjp1)

    def blk_loop(bp, acc):
        h1 = fetch(bp, bufs1, sem1)
        drain(h1)
        acc = compute(bufs1, acc)
        return acc

    acc = lax.fori_loop(0, _NBLK, blk_loop, zero)

    acc_v[...] = acc
    pltpu.sync_copy(acc_v, out_hbm.at[wid])


def _poly_loss_sc(fm, s, idx_i, idx_j):
    mesh = plsc.VectorSubcoreMesh(core_axis_name="c", subcore_axis_name="s")
    row_buf = pltpu.VMEM((_NB, _C), jnp.float32)
    call = pl.kernel(
        _sc_body,
        out_type=jax.ShapeDtypeStruct((_NW, _L), jnp.float32),
        mesh=mesh,
        scratch_types=[
            pltpu.VMEM((_PER_W,), jnp.int32),
            pltpu.VMEM((_PER_W,), jnp.int32),
            pltpu.VMEM((_PER_W,), jnp.int32),
            pltpu.VMEM((_PER_W,), jnp.int32),
            row_buf, row_buf, row_buf, row_buf,
            row_buf, row_buf, row_buf, row_buf,
            pltpu.VMEM((_L,), jnp.float32),
            pltpu.SemaphoreType.DMA,
            pltpu.SemaphoreType.DMA,
        ],
        compiler_params=pltpu.CompilerParams(
            needs_layout_passes=False,
        ),
    )
    return call(fm, s, idx_i, idx_j)


def kernel(input, target, idx_i, idx_j):
    c = input.shape[1]
    fm = input.reshape(c, -1).T
    s = target.reshape(c, -1).T
    partials = _poly_loss_sc(fm, s, idx_i, idx_j)
    n = idx_i.shape[0] // 2
    return partials.sum() / jnp.float32(n) / jnp.float32(c * c)


# speedup vs baseline: 1.1507x; 1.1064x over previous
import jax
import jax.numpy as jnp
from jax import lax
from jax.experimental import pallas as pl
from jax.experimental.pallas import tpu as pltpu
from jax.experimental.pallas import tpu_sc as plsc

_C = 256
_HW = 4096
_N = 32768
_NC = 2
_NS = 16
_NW = _NC * _NS
_L = 16
_PER_W = _N // _NW
_NB = 32
_NBLK = _PER_W // _NB
_UNROLL = 8


def _sc_body(fm_hbm, s_hbm, ii_hbm, ij_hbm, out_hbm,
             i_v, ip_v, j_v, jp_v,
             fi3, fip3, sj3, sjp3,
             acc_v, sem2):
    cid = lax.axis_index("c")
    sid = lax.axis_index("s")
    wid = sid * _NC + cid
    base = wid * _PER_W

    pltpu.sync_copy(ii_hbm.at[pl.ds(base, _PER_W)], i_v)
    pltpu.sync_copy(ii_hbm.at[pl.ds(_N + base, _PER_W)], ip_v)
    pltpu.sync_copy(ij_hbm.at[pl.ds(base, _PER_W)], j_v)
    pltpu.sync_copy(ij_hbm.at[pl.ds(_N + base, _PER_W)], jp_v)

    lanes = lax.iota(jnp.int32, _L)
    zero = jnp.zeros((_L,), jnp.float32)

    def fetch(blk, par):
        off = blk * _NB
        sem = sem2.at[par]
        pltpu.async_copy(fm_hbm.at[i_v.at[pl.ds(off, _NB)]], fi3.at[par], sem)
        pltpu.async_copy(fm_hbm.at[ip_v.at[pl.ds(off, _NB)]], fip3.at[par], sem)
        pltpu.async_copy(s_hbm.at[j_v.at[pl.ds(off, _NB)]], sj3.at[par], sem)
        pltpu.async_copy(s_hbm.at[jp_v.at[pl.ds(off, _NB)]], sjp3.at[par], sem)

    def drain(par):
        sem = sem2.at[par]
        dummy = fm_hbm.at[pl.ds(0, _NB)]
        pltpu.make_async_copy(dummy, fi3.at[par], sem).wait()
        pltpu.make_async_copy(dummy, fip3.at[par], sem).wait()
        pltpu.make_async_copy(dummy, sj3.at[par], sem).wait()
        pltpu.make_async_copy(dummy, sjp3.at[par], sem).wait()

    def compute(par, acc):
        pvec = jnp.full((_L,), par, jnp.int32)
        for g in range(_NB // _L):
            rows = g * _L + lanes

            def chan(k, carry):
                p1, p2, p3, p4 = carry
                cbase = k * _UNROLL
                for _u in range(_UNROLL):
                    cols = jnp.full((_L,), cbase + _u, jnp.int32)
                    a = plsc.load_gather(fi3, [pvec, rows, cols])
                    bb = plsc.load_gather(fip3, [pvec, rows, cols])
                    c_ = plsc.load_gather(sj3, [pvec, rows, cols])
                    d_ = plsc.load_gather(sjp3, [pvec, rows, cols])
                    p1 = p1 + a * bb
                    p2 = p2 + c_ * d_
                    p3 = p3 + a * d_
                    p4 = p4 + c_ * bb
                return p1, p2, p3, p4

            init = (zero, zero, zero, zero)
            p1, p2, p3, p4 = lax.fori_loop(0, _C // _UNROLL, chan, init)
            acc = acc + (p1 * p1 + p2 * p2 - p3 * p3 - p4 * p4)
        return acc

    fetch(0, 0)

    def blk_loop(b, acc):
        par = lax.rem(b, 2)
        fetch(jnp.minimum(b + 1, _NBLK - 1), lax.rem(b + 1, 2))
        drain(par)
        return compute(par, acc)

    acc = lax.fori_loop(0, _NBLK, blk_loop, zero)
    drain(lax.rem(jnp.int32(_NBLK), 2))

    acc_v[...] = acc
    pltpu.sync_copy(acc_v, out_hbm.at[wid])


def _poly_loss_sc(fm, s, idx_i, idx_j):
    mesh = plsc.VectorSubcoreMesh(core_axis_name="c", subcore_axis_name="s")
    ring_buf = pltpu.VMEM((2, _NB, _C), jnp.float32)
    call = pl.kernel(
        _sc_body,
        out_type=jax.ShapeDtypeStruct((_NW, _L), jnp.float32),
        mesh=mesh,
        scratch_types=[
            pltpu.VMEM((_PER_W,), jnp.int32),
            pltpu.VMEM((_PER_W,), jnp.int32),
            pltpu.VMEM((_PER_W,), jnp.int32),
            pltpu.VMEM((_PER_W,), jnp.int32),
            ring_buf, ring_buf, ring_buf, ring_buf,
            pltpu.VMEM((_L,), jnp.float32),
            pltpu.SemaphoreType.DMA((2,)),
        ],
        compiler_params=pltpu.CompilerParams(
            needs_layout_passes=False,
        ),
    )
    return call(fm, s, idx_i, idx_j)


def kernel(input, target, idx_i, idx_j):
    c = input.shape[1]
    fm = input.reshape(c, -1).T
    s = target.reshape(c, -1).T
    partials = _poly_loss_sc(fm, s, idx_i, idx_j)
    n = idx_i.shape[0] // 2
    return partials.sum() / jnp.float32(n) / jnp.float32(c * c)


# speedup vs baseline: 1.2587x; 1.0939x over previous
import jax
import jax.numpy as jnp
from jax import lax
from jax.experimental import pallas as pl
from jax.experimental.pallas import tpu as pltpu
from jax.experimental.pallas import tpu_sc as plsc

_C = 256
_HW = 4096
_N = 32768
_NC = 2
_NS = 16
_L = 16
_CPT = _C // (_NC * _NS)
_CH = 512
_NCHUNK = _N // _CH
_GPC = _CH // _L
_NROW = _N // _L
_SROW = 4 * _GPC


def _sc_body(fmr_hbm, sr_hbm, ii_hbm, ij_hbm, out_hbm,
             tblf_v, tbls_v, ii_v, ip_v, jj_v, jp_v,
             stage_v, sidx_v, acc_sh, sem):
    cid = lax.axis_index("c")
    sid = lax.axis_index("s")
    cb = cid * _NS + sid

    pltpu.sync_copy(fmr_hbm.at[cb], tblf_v)
    pltpu.sync_copy(sr_hbm.at[cb], tbls_v)

    lanes = lax.iota(jnp.int32, _L)
    zero = jnp.zeros((_L,), jnp.float32)

    def zstage(r, _):
        stage_v[r, :] = zero
        return 0

    lax.fori_loop(0, _SROW, zstage, 0)
    stripe = 4 * _NROW // _NS
    for q in range(stripe // _SROW):
        pltpu.sync_copy(stage_v, acc_sh.at[pl.ds(sid * stripe + q * _SROW, _SROW)])
    plsc.subcore_barrier()

    def chunk_loop(ch, _):
        off = ch * _CH
        pltpu.sync_copy(ii_hbm.at[pl.ds(off, _CH)], ii_v)
        pltpu.sync_copy(ii_hbm.at[pl.ds(_N + off, _CH)], ip_v)
        pltpu.sync_copy(ij_hbm.at[pl.ds(off, _CH)], jj_v)
        pltpu.sync_copy(ij_hbm.at[pl.ds(_N + off, _CH)], jp_v)

        def group(g, _):
            gb = g * _L
            i_vec = ii_v[pl.ds(gb, _L)]
            ip_vec = ip_v[pl.ds(gb, _L)]
            j_vec = jj_v[pl.ds(gb, _L)]
            jp_vec = jp_v[pl.ds(gb, _L)]
            p1 = zero
            p2 = zero
            p3 = zero
            p4 = zero
            for c in range(_CPT):
                colv = jnp.full((_L,), c, jnp.int32)
                a = plsc.load_gather(tblf_v, [i_vec, colv])
                b = plsc.load_gather(tblf_v, [ip_vec, colv])
                cs = plsc.load_gather(tbls_v, [j_vec, colv])
                ds = plsc.load_gather(tbls_v, [jp_vec, colv])
                p1 = p1 + a * b
                p2 = p2 + cs * ds
                p3 = p3 + a * ds
                p4 = p4 + cs * b
            stage_v[g, :] = p1
            stage_v[_GPC + g, :] = p2
            stage_v[2 * _GPC + g, :] = p3
            stage_v[3 * _GPC + g, :] = p4
            return 0

        lax.fori_loop(0, _GPC, group, 0)

        base = ch * _GPC
        for v in range(_SROW // _L):
            role = v // 2
            g0 = (v % 2) * _L
            sidx_v[pl.ds(v * _L, _L)] = jnp.full(
                (_L,), role * _NROW + g0 + base, jnp.int32) + lanes

        pltpu.sync_copy(stage_v, acc_sh.at[sidx_v], add=True)
        return 0

    lax.fori_loop(0, _NCHUNK, chunk_loop, 0)
    plsc.subcore_barrier()

    @pl.when(sid == 0)
    def _dump():
        pltpu.sync_copy(acc_sh, out_hbm.at[cid])


def _poly_loss_sc(fmr, sr, idx_i, idx_j):
    mesh = plsc.VectorSubcoreMesh(core_axis_name="c", subcore_axis_name="s")
    call = pl.kernel(
        _sc_body,
        out_type=jax.ShapeDtypeStruct((_NC, 4 * _NROW, _L), jnp.float32),
        mesh=mesh,
        scratch_types=[
            pltpu.VMEM((_HW, _CPT), jnp.float32),
            pltpu.VMEM((_HW, _CPT), jnp.float32),
            pltpu.VMEM((_CH,), jnp.int32),
            pltpu.VMEM((_CH,), jnp.int32),
            pltpu.VMEM((_CH,), jnp.int32),
            pltpu.VMEM((_CH,), jnp.int32),
            pltpu.VMEM((_SROW, _L), jnp.float32),
            pltpu.VMEM((_SROW,), jnp.int32),
            pltpu.VMEM_SHARED((4 * _NROW, _L), jnp.float32),
            pltpu.SemaphoreType.DMA,
        ],
        compiler_params=pltpu.CompilerParams(
            needs_layout_passes=False,
            use_tc_tiling_on_sc=False,
        ),
    )
    return call(fmr, sr, idx_i, idx_j)


def _combine_body(h_ref, o_ref):
    x = h_ref[...]
    p = x[0] + x[1]
    p1 = p[0 * _NROW:1 * _NROW]
    p2 = p[1 * _NROW:2 * _NROW]
    p3 = p[2 * _NROW:3 * _NROW]
    p4 = p[3 * _NROW:4 * _NROW]
    q = p1 * p1 + p2 * p2 - p3 * p3 - p4 * p4
    o_ref[0, 0] = jnp.sum(q)


def _combine(halves):
    return pl.pallas_call(
        _combine_body,
        out_shape=jax.ShapeDtypeStruct((1, 1), jnp.float32),
        in_specs=[pl.BlockSpec(memory_space=pltpu.VMEM)],
        out_specs=pl.BlockSpec(memory_space=pltpu.SMEM),
    )(halves)


def kernel(input, target, idx_i, idx_j):
    c = input.shape[1]
    fmr = input.reshape(c, -1).reshape(_NC * _NS, _CPT, _HW).transpose(0, 2, 1)
    sr = target.reshape(c, -1).reshape(_NC * _NS, _CPT, _HW).transpose(0, 2, 1)
    halves = _poly_loss_sc(fmr, sr, idx_i, idx_j)
    total = _combine(halves)
    n = idx_i.shape[0] // 2
    return total[0, 0] / jnp.float32(n) / jnp.float32(c * c)


# speedup vs baseline: 2.2923x; 1.8211x over previous
import jax
import jax.numpy as jnp
from jax import lax
from jax.experimental import pallas as pl
from jax.experimental.pallas import tpu as pltpu
from jax.experimental.pallas import tpu_sc as plsc

_C = 256
_HW = 4096
_N = 32768
_NC = 2
_NS = 16
_NT = _NC * _NS
_L = 16
_CPT = _C // _NT
_CH = 2048
_NCHUNK = _N // _CH
_GPC = _CH // _L
_NROW = _N // _L


def _sc_body(fmr_hbm, sr_hbm, ii_hbm, ij_hbm, out_hbm,
             tblf_v, tbls_v, ii_v, ip_v, jj_v, jp_v,
             pacc_v, isem, osem):
    cid = lax.axis_index("c")
    sid = lax.axis_index("s")
    cb = cid * _NS + sid

    pltpu.sync_copy(fmr_hbm.at[pl.ds(cb * _CPT, _CPT)], tblf_v)
    pltpu.sync_copy(sr_hbm.at[pl.ds(cb * _CPT, _CPT)], tbls_v)

    zero = jnp.zeros((_L,), jnp.float32)

    def fetch_idx(ch, par):
        off = ch * _CH
        sem = isem.at[par]
        pltpu.async_copy(ii_hbm.at[pl.ds(off, _CH)], ii_v.at[par], sem)
        pltpu.async_copy(ii_hbm.at[pl.ds(_N + off, _CH)], ip_v.at[par], sem)
        pltpu.async_copy(ij_hbm.at[pl.ds(off, _CH)], jj_v.at[par], sem)
        pltpu.async_copy(ij_hbm.at[pl.ds(_N + off, _CH)], jp_v.at[par], sem)

    def drain_idx(par):
        sem = isem.at[par]
        dummy = ii_hbm.at[pl.ds(0, _CH)]
        pltpu.make_async_copy(dummy, ii_v.at[par], sem).wait()
        pltpu.make_async_copy(dummy, ip_v.at[par], sem).wait()
        pltpu.make_async_copy(dummy, jj_v.at[par], sem).wait()
        pltpu.make_async_copy(dummy, jp_v.at[par], sem).wait()

    def put_out(ch, par):
        sem = osem.at[par]
        rows = pl.ds(ch * _GPC, _GPC)
        for role in range(4):
            pltpu.async_copy(pacc_v.at[par, role],
                             out_hbm.at[role, cb, rows], sem)

    def drain_out(par):
        sem = osem.at[par]
        dummy = out_hbm.at[0, 0, pl.ds(0, _GPC)]
        for role in range(4):
            pltpu.make_async_copy(dummy, pacc_v.at[par, role], sem).wait()

    fetch_idx(0, 0)
    fetch_idx(1, 1)

    def chunk_loop(ch, _):
        par = lax.rem(ch, 2)
        drain_idx(par)

        @pl.when(ch >= 2)
        def _():
            drain_out(par)

        def group(g, _):
            gb = g * _L
            i_vec = ii_v[par, pl.ds(gb, _L)]
            ip_vec = ip_v[par, pl.ds(gb, _L)]
            j_vec = jj_v[par, pl.ds(gb, _L)]
            jp_vec = jp_v[par, pl.ds(gb, _L)]
            p1 = zero
            p2 = zero
            p3 = zero
            p4 = zero
            for c in range(_CPT):
                colv = jnp.full((_L,), c, jnp.int32)
                a = plsc.load_gather(tblf_v, [colv, i_vec])
                b = plsc.load_gather(tblf_v, [colv, ip_vec])
                cs = plsc.load_gather(tbls_v, [colv, j_vec])
                ds_ = plsc.load_gather(tbls_v, [colv, jp_vec])
                p1 = p1 + a * b
                p2 = p2 + cs * ds_
                p3 = p3 + a * ds_
                p4 = p4 + cs * b
            pacc_v[par, 0, g, :] = p1
            pacc_v[par, 1, g, :] = p2
            pacc_v[par, 2, g, :] = p3
            pacc_v[par, 3, g, :] = p4
            return 0

        lax.fori_loop(0, _GPC, group, 0)
        put_out(ch, par)

        @pl.when(ch < _NCHUNK - 2)
        def _():
            fetch_idx(ch + 2, par)

        return 0

    lax.fori_loop(0, _NCHUNK, chunk_loop, 0)
    drain_out(0)
    drain_out(1)


def _poly_loss_sc(fmr, sr, idx_i, idx_j):
    mesh = plsc.VectorSubcoreMesh(core_axis_name="c", subcore_axis_name="s")
    call = pl.kernel(
        _sc_body,
        out_type=jax.ShapeDtypeStruct((4, _NT, _NROW, _L), jnp.float32),
        mesh=mesh,
        scratch_types=[
            pltpu.VMEM((_CPT, _HW), jnp.float32),
            pltpu.VMEM((_CPT, _HW), jnp.float32),
            pltpu.VMEM((2, _CH), jnp.int32),
            pltpu.VMEM((2, _CH), jnp.int32),
            pltpu.VMEM((2, _CH), jnp.int32),
            pltpu.VMEM((2, _CH), jnp.int32),
            pltpu.VMEM((2, 4, _GPC, _L), jnp.float32),
            pltpu.SemaphoreType.DMA((2,)),
            pltpu.SemaphoreType.DMA((2,)),
        ],
        compiler_params=pltpu.CompilerParams(
            needs_layout_passes=False,
            use_tc_tiling_on_sc=False,
        ),
    )
    return call(fmr, sr, idx_i, idx_j)


_TCBLK = 2048


def _combine_body(p_ref, o_ref):
    k = pl.program_id(0)
    x = p_ref[...]
    s = jnp.sum(x, axis=1)
    q = s * s
    psum = jnp.sum(q[0:2]) - jnp.sum(q[2:4])

    @pl.when(k == 0)
    def _():
        o_ref[0, 0] = psum

    @pl.when(k != 0)
    def _():
        o_ref[0, 0] += psum


def _combine(p):
    grid = _N // _TCBLK
    return pl.pallas_call(
        _combine_body,
        grid=(grid,),
        in_specs=[pl.BlockSpec((4, _NT, _TCBLK), lambda k: (0, 0, k))],
        out_specs=pl.BlockSpec(memory_space=pltpu.SMEM),
        out_shape=jax.ShapeDtypeStruct((1, 1), jnp.float32),
    )(p)


def kernel(input, target, idx_i, idx_j):
    c = input.shape[1]
    fmr = input.reshape(c, -1)
    sr = target.reshape(c, -1)
    p = _poly_loss_sc(fmr, sr, idx_i, idx_j)
    total = _combine(p.reshape(4, _NT, _N))
    n = idx_i.shape[0] // 2
    return total[0, 0] / jnp.float32(n) / jnp.float32(c * c)


# speedup vs baseline: 4.1268x; 1.8003x over previous
import jax
import jax.numpy as jnp
from jax import lax
from jax.experimental import pallas as pl
from jax.experimental.pallas import tpu as pltpu
from jax.experimental.pallas import tpu_sc as plsc

_C = 256
_HW = 4096
_N = 32768
_NC = 2
_NS = 16
_NT = _NC * _NS
_L = 16
_CPT = _C // _NT
_CH = 2048
_NCHUNK = _N // _CH
_GPC = _CH // _L
_NROW = _N // _L


def _sc_body(fmr_hbm, sr_hbm, ii_hbm, ij_hbm, out_hbm,
             tblf_v, tbls_v, ii_v, ip_v, jj_v, jp_v,
             pacc_v, isem, osem):
    cid = lax.axis_index("c")
    sid = lax.axis_index("s")
    cb = cid * _NS + sid

    pltpu.sync_copy(fmr_hbm.at[pl.ds(cb * _CPT, _CPT)], tblf_v)
    pltpu.sync_copy(sr_hbm.at[pl.ds(cb * _CPT, _CPT)], tbls_v)

    zero = jnp.zeros((_L,), jnp.float32)

    def fetch_idx(ch, par):
        off = ch * _CH
        sem = isem.at[par]
        pltpu.async_copy(ii_hbm.at[pl.ds(off, _CH)], ii_v.at[par], sem)
        pltpu.async_copy(ii_hbm.at[pl.ds(_N + off, _CH)], ip_v.at[par], sem)
        pltpu.async_copy(ij_hbm.at[pl.ds(off, _CH)], jj_v.at[par], sem)
        pltpu.async_copy(ij_hbm.at[pl.ds(_N + off, _CH)], jp_v.at[par], sem)

    def drain_idx(par):
        sem = isem.at[par]
        dummy = ii_hbm.at[pl.ds(0, _CH)]
        pltpu.make_async_copy(dummy, ii_v.at[par], sem).wait()
        pltpu.make_async_copy(dummy, ip_v.at[par], sem).wait()
        pltpu.make_async_copy(dummy, jj_v.at[par], sem).wait()
        pltpu.make_async_copy(dummy, jp_v.at[par], sem).wait()

    def put_out(ch, par):
        sem = osem.at[par]
        cols = pl.ds(ch * _CH, _CH)
        for role in range(4):
            pltpu.async_copy(pacc_v.at[par, role],
                             out_hbm.at[role, cb, cols], sem)

    def drain_out(par):
        sem = osem.at[par]
        dummy = out_hbm.at[0, 0, pl.ds(0, _CH)]
        for role in range(4):
            pltpu.make_async_copy(dummy, pacc_v.at[par, role], sem).wait()

    fetch_idx(0, 0)
    fetch_idx(1, 1)

    def chunk_loop(ch, _):
        par = lax.rem(ch, 2)
        drain_idx(par)

        @pl.when(ch >= 2)
        def _():
            drain_out(par)

        def group(g, _):
            gb = g * _L
            i_vec = ii_v[par, pl.ds(gb, _L)]
            ip_vec = ip_v[par, pl.ds(gb, _L)]
            j_vec = jj_v[par, pl.ds(gb, _L)]
            jp_vec = jp_v[par, pl.ds(gb, _L)]
            p1 = zero
            p2 = zero
            p3 = zero
            p4 = zero
            for c in range(_CPT):
                colv = jnp.full((_L,), c, jnp.int32)
                a = plsc.load_gather(tblf_v, [colv, i_vec])
                b = plsc.load_gather(tblf_v, [colv, ip_vec])
                cs = plsc.load_gather(tbls_v, [colv, j_vec])
                ds_ = plsc.load_gather(tbls_v, [colv, jp_vec])
                p1 = p1 + a * b
                p2 = p2 + cs * ds_
                p3 = p3 + a * ds_
                p4 = p4 + cs * b
            pacc_v[par, 0, pl.ds(gb, _L)] = p1
            pacc_v[par, 1, pl.ds(gb, _L)] = p2
            pacc_v[par, 2, pl.ds(gb, _L)] = p3
            pacc_v[par, 3, pl.ds(gb, _L)] = p4
            return 0

        lax.fori_loop(0, _GPC, group, 0)
        put_out(ch, par)

        @pl.when(ch < _NCHUNK - 2)
        def _():
            fetch_idx(ch + 2, par)

        return 0

    lax.fori_loop(0, _NCHUNK, chunk_loop, 0)
    drain_out(0)
    drain_out(1)


def _poly_loss_sc(fmr, sr, idx_i, idx_j):
    mesh = plsc.VectorSubcoreMesh(core_axis_name="c", subcore_axis_name="s")
    call = pl.kernel(
        _sc_body,
        out_type=jax.ShapeDtypeStruct((4, _NT, _N), jnp.float32),
        mesh=mesh,
        scratch_types=[
            pltpu.VMEM((_CPT, _HW), jnp.float32),
            pltpu.VMEM((_CPT, _HW), jnp.float32),
            pltpu.VMEM((2, _CH), jnp.int32),
            pltpu.VMEM((2, _CH), jnp.int32),
            pltpu.VMEM((2, _CH), jnp.int32),
            pltpu.VMEM((2, _CH), jnp.int32),
            pltpu.VMEM((2, 4, _CH), jnp.float32),
            pltpu.SemaphoreType.DMA((2,)),
            pltpu.SemaphoreType.DMA((2,)),
        ],
        compiler_params=pltpu.CompilerParams(
            needs_layout_passes=False,
            use_tc_tiling_on_sc=False,
        ),
    )
    return call(fmr, sr, idx_i, idx_j)


_TCBLK = 2048


def _combine_body(p_ref, o_ref):
    k = pl.program_id(0)
    x = p_ref[...]
    s = jnp.sum(x, axis=1)
    q = s * s
    psum = jnp.sum(q[0:2]) - jnp.sum(q[2:4])

    @pl.when(k == 0)
    def _():
        o_ref[0, 0] = psum

    @pl.when(k != 0)
    def _():
        o_ref[0, 0] += psum


def _combine(p):
    grid = _N // _TCBLK
    return pl.pallas_call(
        _combine_body,
        grid=(grid,),
        in_specs=[pl.BlockSpec((4, _NT, _TCBLK), lambda k: (0, 0, k))],
        out_specs=pl.BlockSpec(memory_space=pltpu.SMEM),
        out_shape=jax.ShapeDtypeStruct((1, 1), jnp.float32),
    )(p)


def kernel(input, target, idx_i, idx_j):
    c = input.shape[1]
    fmr = input.reshape(c, -1)
    sr = target.reshape(c, -1)
    p = _poly_loss_sc(fmr, sr, idx_i, idx_j)
    total = _combine(p)
    n = idx_i.shape[0] // 2
    return total[0, 0] / jnp.float32(n) / jnp.float32(c * c)


# speedup vs baseline: 4.2526x; 1.0305x over previous
import jax
import jax.numpy as jnp
from jax import lax
from jax.experimental import pallas as pl
from jax.experimental.pallas import tpu as pltpu
from jax.experimental.pallas import tpu_sc as plsc

_C = 256
_HW = 4096
_N = 32768
_NC = 2
_NS = 16
_NT = _NC * _NS
_L = 16
_CPT = _C // _NT
_CH = 2048
_NCHUNK = _N // _CH
_GPC = _CH // _L
_NROW = _N // _L


def _sc_body(fmr_hbm, sr_hbm, ii_hbm, ij_hbm, out_hbm,
             tblf_v, tbls_v, ii_v, ip_v, jj_v, jp_v,
             pacc_v, isem, osem):
    cid = lax.axis_index("c")
    sid = lax.axis_index("s")
    cb = cid * _NS + sid

    pltpu.sync_copy(fmr_hbm.at[pl.ds(cb * _CPT, _CPT)], tblf_v)
    pltpu.sync_copy(sr_hbm.at[pl.ds(cb * _CPT, _CPT)], tbls_v)

    zero = jnp.zeros((_L,), jnp.float32)

    def fetch_idx(ch, par):
        off = ch * _CH
        sem = isem.at[par]
        pltpu.async_copy(ii_hbm.at[pl.ds(off, _CH)], ii_v.at[par], sem)
        pltpu.async_copy(ii_hbm.at[pl.ds(_N + off, _CH)], ip_v.at[par], sem)
        pltpu.async_copy(ij_hbm.at[pl.ds(off, _CH)], jj_v.at[par], sem)
        pltpu.async_copy(ij_hbm.at[pl.ds(_N + off, _CH)], jp_v.at[par], sem)

    def drain_idx(par):
        sem = isem.at[par]
        dummy = ii_hbm.at[pl.ds(0, _CH)]
        pltpu.make_async_copy(dummy, ii_v.at[par], sem).wait()
        pltpu.make_async_copy(dummy, ip_v.at[par], sem).wait()
        pltpu.make_async_copy(dummy, jj_v.at[par], sem).wait()
        pltpu.make_async_copy(dummy, jp_v.at[par], sem).wait()

    def put_out(ch, par):
        sem = osem.at[par]
        cols = pl.ds(ch * _CH, _CH)
        for role in range(4):
            pltpu.async_copy(pacc_v.at[par, role],
                             out_hbm.at[role, cb, cols], sem)

    def drain_out(par):
        sem = osem.at[par]
        dummy = out_hbm.at[0, 0, pl.ds(0, _CH)]
        for role in range(4):
            pltpu.make_async_copy(dummy, pacc_v.at[par, role], sem).wait()

    fetch_idx(0, 0)
    fetch_idx(1, 1)

    def chunk_loop(ch, _):
        par = lax.rem(ch, 2)
        drain_idx(par)

        @pl.when(ch >= 2)
        def _():
            drain_out(par)

        def group(g, _):
            gb = g * _L
            i_vec = ii_v[par, pl.ds(gb, _L)]
            ip_vec = ip_v[par, pl.ds(gb, _L)]
            j_vec = jj_v[par, pl.ds(gb, _L)]
            jp_vec = jp_v[par, pl.ds(gb, _L)]
            p1 = zero
            p2 = zero
            p3 = zero
            p4 = zero
            for c in range(_CPT):
                colv = jnp.full((_L,), c, jnp.int32)
                a = plsc.load_gather(tblf_v, [colv, i_vec])
                b = plsc.load_gather(tblf_v, [colv, ip_vec])
                cs = plsc.load_gather(tbls_v, [colv, j_vec])
                ds_ = plsc.load_gather(tbls_v, [colv, jp_vec])
                p1 = p1 + a * b
                p2 = p2 + cs * ds_
                p3 = p3 + a * ds_
                p4 = p4 + cs * b
            pacc_v[par, 0, pl.ds(gb, _L)] = p1
            pacc_v[par, 1, pl.ds(gb, _L)] = p2
            pacc_v[par, 2, pl.ds(gb, _L)] = p3
            pacc_v[par, 3, pl.ds(gb, _L)] = p4
            return 0

        lax.fori_loop(0, _GPC, group, 0)
        put_out(ch, par)

        @pl.when(ch < _NCHUNK - 2)
        def _():
            fetch_idx(ch + 2, par)

        return 0

    lax.fori_loop(0, _NCHUNK, chunk_loop, 0)
    drain_out(0)
    drain_out(1)


def _poly_loss_sc(fmr, sr, idx_i, idx_j):
    mesh = plsc.VectorSubcoreMesh(core_axis_name="c", subcore_axis_name="s")
    call = pl.kernel(
        _sc_body,
        out_type=jax.ShapeDtypeStruct((4, _NT, _N), jnp.float32),
        mesh=mesh,
        scratch_types=[
            pltpu.VMEM((_CPT, _HW), jnp.float32),
            pltpu.VMEM((_CPT, _HW), jnp.float32),
            pltpu.VMEM((2, _CH), jnp.int32),
            pltpu.VMEM((2, _CH), jnp.int32),
            pltpu.VMEM((2, _CH), jnp.int32),
            pltpu.VMEM((2, _CH), jnp.int32),
            pltpu.VMEM((2, 4, _CH), jnp.float32),
            pltpu.SemaphoreType.DMA((2,)),
            pltpu.SemaphoreType.DMA((2,)),
        ],
        compiler_params=pltpu.CompilerParams(
            needs_layout_passes=False,
            use_tc_tiling_on_sc=False,
            disable_bounds_checks=True,
        ),
    )
    return call(fmr, sr, idx_i, idx_j)


_TCBLK = 4096


def _combine_body(p_ref, o_ref):
    k = pl.program_id(0)
    x = p_ref[...]
    s = jnp.sum(x, axis=1)
    q = s * s
    psum = jnp.sum(q[0:2]) - jnp.sum(q[2:4])

    @pl.when(k == 0)
    def _():
        o_ref[0, 0] = psum

    @pl.when(k != 0)
    def _():
        o_ref[0, 0] += psum


def _combine(p):
    grid = _N // _TCBLK
    return pl.pallas_call(
        _combine_body,
        grid=(grid,),
        in_specs=[pl.BlockSpec((4, _NT, _TCBLK), lambda k: (0, 0, k))],
        out_specs=pl.BlockSpec(memory_space=pltpu.SMEM),
        out_shape=jax.ShapeDtypeStruct((1, 1), jnp.float32),
    )(p)


def kernel(input, target, idx_i, idx_j):
    c = input.shape[1]
    fmr = input.reshape(c, -1)
    sr = target.reshape(c, -1)
    p = _poly_loss_sc(fmr, sr, idx_i, idx_j)
    total = _combine(p)
    n = idx_i.shape[0] // 2
    return total[0, 0] / jnp.float32(n) / jnp.float32(c * c)


# speedup vs baseline: 4.6916x; 1.1032x over previous
import jax
import jax.numpy as jnp
from jax import lax
from jax.experimental import pallas as pl
from jax.experimental.pallas import tpu as pltpu
from jax.experimental.pallas import tpu_sc as plsc

_C = 256
_HW = 4096
_N = 32768
_NC = 2
_NS = 16
_NT = _NC * _NS
_L = 16
_CPT = _C // _NT
_CH = 2048
_NCHUNK = _N // _CH
_GPC = _CH // _L
_NROW = _N // _L


def _sc_body(fmr_hbm, sr_hbm, ii_hbm, ij_hbm, out_hbm,
             tblf_v, tbls_v, ii_v, ip_v, jj_v, jp_v,
             pacc_v, isem, osem):
    cid = lax.axis_index("c")
    sid = lax.axis_index("s")
    cb = cid * _NS + sid

    pltpu.sync_copy(fmr_hbm.at[pl.ds(cb * _CPT, _CPT)], tblf_v)
    pltpu.sync_copy(sr_hbm.at[pl.ds(cb * _CPT, _CPT)], tbls_v)

    zero = jnp.zeros((_L,), jnp.float32)

    def fetch_idx(ch, par):
        off = ch * _CH
        sem = isem.at[par]
        pltpu.async_copy(ii_hbm.at[pl.ds(off, _CH)], ii_v.at[par], sem)
        pltpu.async_copy(ii_hbm.at[pl.ds(_N + off, _CH)], ip_v.at[par], sem)
        pltpu.async_copy(ij_hbm.at[pl.ds(off, _CH)], jj_v.at[par], sem)
        pltpu.async_copy(ij_hbm.at[pl.ds(_N + off, _CH)], jp_v.at[par], sem)

    def drain_idx(par):
        sem = isem.at[par]
        dummy = ii_hbm.at[pl.ds(0, _CH)]
        pltpu.make_async_copy(dummy, ii_v.at[par], sem).wait()
        pltpu.make_async_copy(dummy, ip_v.at[par], sem).wait()
        pltpu.make_async_copy(dummy, jj_v.at[par], sem).wait()
        pltpu.make_async_copy(dummy, jp_v.at[par], sem).wait()

    def put_out(ch, par):
        sem = osem.at[par]
        cols = pl.ds(ch * _CH, _CH)
        for role in range(4):
            pltpu.async_copy(pacc_v.at[par, role],
                             out_hbm.at[role, cb, cols], sem)

    def drain_out(par):
        sem = osem.at[par]
        dummy = out_hbm.at[0, 0, pl.ds(0, _CH)]
        for role in range(4):
            pltpu.make_async_copy(dummy, pacc_v.at[par, role], sem).wait()

    fetch_idx(0, 0)
    fetch_idx(1, 1)

    def chunk_loop(ch, _):
        par = lax.rem(ch, 2)
        drain_idx(par)

        @pl.when(ch >= 2)
        def _():
            drain_out(par)

        @plsc.parallel_loop(0, _GPC, 1, unroll=2)
        def group(g):
            gb = g * _L
            i_vec = ii_v[par, pl.ds(gb, _L)]
            ip_vec = ip_v[par, pl.ds(gb, _L)]
            j_vec = jj_v[par, pl.ds(gb, _L)]
            jp_vec = jp_v[par, pl.ds(gb, _L)]
            p1 = zero
            p2 = zero
            p3 = zero
            p4 = zero
            for c in range(_CPT):
                colv = jnp.full((_L,), c, jnp.int32)
                a = plsc.load_gather(tblf_v, [colv, i_vec])
                b = plsc.load_gather(tblf_v, [colv, ip_vec])
                cs = plsc.load_gather(tbls_v, [colv, j_vec])
                ds_ = plsc.load_gather(tbls_v, [colv, jp_vec])
                p1 = p1 + a * b
                p2 = p2 + cs * ds_
                p3 = p3 + a * ds_
                p4 = p4 + cs * b
            pacc_v[par, 0, pl.ds(gb, _L)] = p1
            pacc_v[par, 1, pl.ds(gb, _L)] = p2
            pacc_v[par, 2, pl.ds(gb, _L)] = p3
            pacc_v[par, 3, pl.ds(gb, _L)] = p4
        put_out(ch, par)

        @pl.when(ch < _NCHUNK - 2)
        def _():
            fetch_idx(ch + 2, par)

        return 0

    lax.fori_loop(0, _NCHUNK, chunk_loop, 0)
    drain_out(0)
    drain_out(1)


def _poly_loss_sc(fmr, sr, idx_i, idx_j):
    mesh = plsc.VectorSubcoreMesh(core_axis_name="c", subcore_axis_name="s")
    call = pl.kernel(
        _sc_body,
        out_type=jax.ShapeDtypeStruct((4, _NT, _N), jnp.float32),
        mesh=mesh,
        scratch_types=[
            pltpu.VMEM((_CPT, _HW), jnp.float32),
            pltpu.VMEM((_CPT, _HW), jnp.float32),
            pltpu.VMEM((2, _CH), jnp.int32),
            pltpu.VMEM((2, _CH), jnp.int32),
            pltpu.VMEM((2, _CH), jnp.int32),
            pltpu.VMEM((2, _CH), jnp.int32),
            pltpu.VMEM((2, 4, _CH), jnp.float32),
            pltpu.SemaphoreType.DMA((2,)),
            pltpu.SemaphoreType.DMA((2,)),
        ],
        compiler_params=pltpu.CompilerParams(
            needs_layout_passes=False,
            use_tc_tiling_on_sc=False,
            disable_bounds_checks=True,
        ),
    )
    return call(fmr, sr, idx_i, idx_j)


_TCBLK = 4096


def _combine_body(p_ref, o_ref):
    k = pl.program_id(0)
    x = p_ref[...]
    s = jnp.sum(x, axis=1)
    q = s * s
    psum = jnp.sum(q[0:2]) - jnp.sum(q[2:4])

    @pl.when(k == 0)
    def _():
        o_ref[0, 0] = psum

    @pl.when(k != 0)
    def _():
        o_ref[0, 0] += psum


def _combine(p):
    grid = _N // _TCBLK
    return pl.pallas_call(
        _combine_body,
        grid=(grid,),
        in_specs=[pl.BlockSpec((4, _NT, _TCBLK), lambda k: (0, 0, k))],
        out_specs=pl.BlockSpec(memory_space=pltpu.SMEM),
        out_shape=jax.ShapeDtypeStruct((1, 1), jnp.float32),
    )(p)


def kernel(input, target, idx_i, idx_j):
    c = input.shape[1]
    fmr = input.reshape(c, -1)
    sr = target.reshape(c, -1)
    p = _poly_loss_sc(fmr, sr, idx_i, idx_j)
    total = _combine(p)
    n = idx_i.shape[0] // 2
    return total[0, 0] / jnp.float32(n) / jnp.float32(c * c)
